# baseline (speedup 1.0000x reference)
import jax
import jax.numpy as jnp
from jax.experimental import pallas as pl
from jax.experimental.pallas import tpu as pltpu

_B, _FL, _ML = 16, 2048, 256
_F, _H, _VE, _V = 128, 128, 32, 64
_NH, _DH, _DFF = 4, 32, 512
_BF = jnp.bfloat16


def _layer_norm(x, g, b):
    mu = jnp.mean(x, axis=-1, keepdims=True)
    d = x - mu
    var = jnp.mean(d * d, axis=-1, keepdims=True)
    return g * (d * jax.lax.rsqrt(var + 1e-5)) + b


def _bdot(a, b):
    return jnp.dot(a.astype(_BF), b.astype(_BF),
                   preferred_element_type=jnp.float32)


def _body(vid_ref, feat_ref, mora_ref, emb_ref, Wpm_ref, bpm_ref, Wpf_ref,
          bpf_ref, Wq_ref, Wk_ref, Wv_ref, Wo_ref, ln1g_ref, ln1b_ref,
          W1_ref, b1_ref, W2_ref, b2_ref, ln2g_ref, ln2b_ref, Wpost_ref,
          bpost_ref, out_ref, wkv_s, ew_s, wq_s, bv_s):
    @pl.when(pl.program_id(0) == 0)
    def _():
        wkv_s[:, :_NH * _DH] = _bdot(Wpf_ref[...], Wk_ref[...]).astype(_BF)
        wkv_s[:, _NH * _DH:] = _bdot(Wpf_ref[...], Wv_ref[...]).astype(_BF)
        ew_s[...] = _bdot(emb_ref[...], Wpm_ref[:_VE, :]).astype(_BF)
        wq_s[...] = (Wq_ref[...] * (1.4426950408889634 / (_DH ** 0.5))
                     ).astype(_BF)
        bv_s[...] = _bdot(bpf_ref[...], Wv_ref[...])

    feat = feat_ref[0].astype(_BF)
    ids = mora_ref[0]
    ohT = (jax.lax.broadcasted_iota(jnp.int32, (_ML, _FL), 0) == ids
           ).astype(_BF)
    cnt = jnp.sum(ohT.astype(jnp.float32), axis=1, keepdims=True)
    ssum = jnp.dot(ohT, feat, preferred_element_type=jnp.float32)
    inv = jnp.where(cnt > 0, 1.0 / jnp.maximum(cnt, 1.0), 0.0)
    mora_feat = ssum * inv

    vids = vid_ref[0]
    voh = (jax.lax.broadcasted_iota(jnp.int32, (_ML, _V), 1) == vids
           ).astype(_BF)
    mhA = jnp.dot(voh, ew_s[...], preferred_element_type=jnp.float32)
    mh = mhA + _bdot(mora_feat, Wpm_ref[_VE:, :]) + bpm_ref[...]

    kv = jnp.dot(feat, wkv_s[...],
                 preferred_element_type=jnp.float32).astype(_BF)
    k = kv[:, :_NH * _DH]
    v = kv[:, _NH * _DH:]
    q = jnp.dot(mh.astype(_BF), wq_s[...],
                preferred_element_type=jnp.float32).astype(_BF)

    ones_col = jnp.ones((_FL, 8), dtype=_BF)
    ctxs = []
    for h_i in range(_NH):
        sl = slice(h_i * _DH, (h_i + 1) * _DH)
        s = jax.lax.dot_general(q[:, sl], k[:, sl], (((1,), (1,)), ((), ())),
                                preferred_element_type=jnp.float32)
        e = jnp.exp2(s.astype(_BF))
        viaug = jnp.concatenate([v[:, sl], ones_col], axis=1)
        cd = jnp.dot(e, viaug, preferred_element_type=jnp.float32)
        ctxs.append(cd[:, :_DH] * (1.0 / cd[:, _DH:_DH + 1]))
    ctx = jnp.concatenate(ctxs, axis=1) + bv_s[...]

    h = mh + _bdot(ctx, Wo_ref[...])
    h = _layer_norm(h, ln1g_ref[...], ln1b_ref[...])
    ff = jnp.maximum(_bdot(h, W1_ref[...]) + b1_ref[...], 0.0)
    h2 = h + _bdot(ff, W2_ref[...]) + b2_ref[...]
    h2 = _layer_norm(h2, ln2g_ref[...], ln2b_ref[...])
    out_ref[0] = _bdot(h2, Wpost_ref[...]) + bpost_ref[...]


def kernel(vowel_ids, features, mora_index, emb, Wpm, bpm, Wpf, bpf, Wq, Wk,
           Wv, Wo, ln1_g, ln1_b, W1, b1, W2, b2, ln2_g, ln2_b, Wpost, bpost):
    B_, FL_, F_ = features.shape
    ML_ = vowel_ids.shape[1]

    vid3 = vowel_ids.astype(jnp.int32).reshape(B_, ML_, 1)
    mora3 = mora_index.astype(jnp.int32).reshape(B_, 1, FL_)
    row = lambda x: x.reshape(1, -1)

    def full(arr):
        return pl.BlockSpec(arr.shape, lambda b: (0,) * arr.ndim)

    weights = [emb, Wpm, row(bpm), Wpf, row(bpf), Wq, Wk, Wv, Wo,
               row(ln1_g), row(ln1_b), W1, row(b1), W2, row(b2),
               row(ln2_g), row(ln2_b), Wpost, row(bpost)]

    out = pl.pallas_call(
        _body,
        grid=(B_,),
        in_specs=[
            pl.BlockSpec((1, ML_, 1), lambda b: (b, 0, 0)),
            pl.BlockSpec((1, FL_, F_), lambda b: (b, 0, 0)),
            pl.BlockSpec((1, 1, FL_), lambda b: (b, 0, 0)),
        ] + [full(w) for w in weights],
        out_specs=pl.BlockSpec((1, ML_, 8), lambda b: (b, 0, 0)),
        out_shape=jax.ShapeDtypeStruct((B_, ML_, 8), jnp.float32),
        scratch_shapes=[
            pltpu.VMEM((F_, 2 * _NH * _DH), _BF),
            pltpu.VMEM((_V, _H), _BF),
            pltpu.VMEM((_H, _NH * _DH), _BF),
            pltpu.VMEM((1, _NH * _DH), jnp.float32),
        ],
    )(vid3, features, mora3, *weights)
    return out.reshape(B_, ML_, 2, 4)


# speedup vs baseline: 1.0228x; 1.0228x over previous
import jax
import jax.numpy as jnp
from jax.experimental import pallas as pl
from jax.experimental.pallas import tpu as pltpu

_B, _FL, _ML = 16, 2048, 256
_F, _H, _VE, _V = 128, 128, 32, 64
_NH, _DH, _DFF = 4, 32, 512
_BF = jnp.bfloat16


def _layer_norm(x, g, b):
    mu = jnp.mean(x, axis=-1, keepdims=True)
    d = x - mu
    var = jnp.mean(d * d, axis=-1, keepdims=True)
    return g * (d * jax.lax.rsqrt(var + 1e-5)) + b


def _bdot(a, b):
    return jnp.dot(a.astype(_BF), b.astype(_BF),
                   preferred_element_type=jnp.float32)


def _body(vid_ref, feat_ref, mora_ref, emb_ref, Wpm_ref, bpm_ref, Wpf_ref,
          bpf_ref, Wq_ref, Wk_ref, Wv_ref, Wo_ref, ln1g_ref, ln1b_ref,
          W1_ref, b1_ref, W2_ref, b2_ref, ln2g_ref, ln2b_ref, Wpost_ref,
          bpost_ref, out_ref, wk_s, wv_s, ew_s, wq_s, bv_s):
    @pl.when(pl.program_id(0) == 0)
    def _():
        wk_s[...] = _bdot(Wpf_ref[...], Wk_ref[...]).astype(_BF)
        wv_s[...] = _bdot(Wpf_ref[...], Wv_ref[...]).astype(_BF)
        ew_s[...] = _bdot(emb_ref[...], Wpm_ref[:_VE, :]).astype(_BF)
        wq_s[...] = (Wq_ref[...] * (1.4426950408889634 / (_DH ** 0.5))
                     ).astype(_BF)
        bv_s[...] = _bdot(bpf_ref[...], Wv_ref[...])

    feat = feat_ref[0].astype(_BF)
    ids = mora_ref[0]
    ohT = (jax.lax.broadcasted_iota(jnp.int32, (_ML, _FL), 0) == ids
           ).astype(_BF)
    cnt = jnp.sum(ohT.astype(jnp.float32), axis=1, keepdims=True)
    ssum = jnp.dot(ohT, feat, preferred_element_type=jnp.float32)
    inv = jnp.where(cnt > 0, 1.0 / jnp.maximum(cnt, 1.0), 0.0)
    mora_feat = ssum * inv

    vids = vid_ref[0]
    voh = (jax.lax.broadcasted_iota(jnp.int32, (_ML, _V), 1) == vids
           ).astype(_BF)
    mhA = jnp.dot(voh, ew_s[...], preferred_element_type=jnp.float32)
    mh = mhA + _bdot(mora_feat, Wpm_ref[_VE:, :]) + bpm_ref[...]

    k = jnp.dot(feat, wk_s[...],
                preferred_element_type=jnp.float32).astype(_BF)
    v = jnp.dot(feat, wv_s[...],
                preferred_element_type=jnp.float32).astype(_BF)
    q = jnp.dot(mh.astype(_BF), wq_s[...],
                preferred_element_type=jnp.float32).astype(_BF)

    ones_col = jnp.ones((_FL, 8), dtype=_BF)
    sls = [slice(h_i * _DH, (h_i + 1) * _DH) for h_i in range(_NH)]
    ss = [jax.lax.dot_general(q[:, sl], k[:, sl], (((1,), (1,)), ((), ())),
                              preferred_element_type=jnp.float32)
          for sl in sls]
    es = [jnp.exp2(s.astype(_BF)) for s in ss]
    vaugs = [jnp.concatenate([v[:, sl], ones_col], axis=1) for sl in sls]
    cds = [jnp.dot(e, va, preferred_element_type=jnp.float32)
           for e, va in zip(es, vaugs)]
    ctxs = [cd[:, :_DH] * (1.0 / cd[:, _DH:_DH + 1]) for cd in cds]
    ctx = jnp.concatenate(ctxs, axis=1) + bv_s[...]

    h = mh + _bdot(ctx, Wo_ref[...])
    h = _layer_norm(h, ln1g_ref[...], ln1b_ref[...])
    ff = jnp.maximum(_bdot(h, W1_ref[...]) + b1_ref[...], 0.0)
    h2 = h + _bdot(ff, W2_ref[...]) + b2_ref[...]
    h2 = _layer_norm(h2, ln2g_ref[...], ln2b_ref[...])
    out_ref[0] = _bdot(h2, Wpost_ref[...]) + bpost_ref[...]


def kernel(vowel_ids, features, mora_index, emb, Wpm, bpm, Wpf, bpf, Wq, Wk,
           Wv, Wo, ln1_g, ln1_b, W1, b1, W2, b2, ln2_g, ln2_b, Wpost, bpost):
    B_, FL_, F_ = features.shape
    ML_ = vowel_ids.shape[1]

    vid3 = vowel_ids.astype(jnp.int32).reshape(B_, ML_, 1)
    mora3 = mora_index.astype(jnp.int32).reshape(B_, 1, FL_)
    row = lambda x: x.reshape(1, -1)

    def full(arr):
        return pl.BlockSpec(arr.shape, lambda b: (0,) * arr.ndim)

    weights = [emb, Wpm, row(bpm), Wpf, row(bpf), Wq, Wk, Wv, Wo,
               row(ln1_g), row(ln1_b), W1, row(b1), W2, row(b2),
               row(ln2_g), row(ln2_b), Wpost, row(bpost)]

    out = pl.pallas_call(
        _body,
        grid=(B_,),
        in_specs=[
            pl.BlockSpec((1, ML_, 1), lambda b: (b, 0, 0)),
            pl.BlockSpec((1, FL_, F_), lambda b: (b, 0, 0)),
            pl.BlockSpec((1, 1, FL_), lambda b: (b, 0, 0)),
        ] + [full(w) for w in weights],
        out_specs=pl.BlockSpec((1, ML_, 8), lambda b: (b, 0, 0)),
        out_shape=jax.ShapeDtypeStruct((B_, ML_, 8), jnp.float32),
        scratch_shapes=[
            pltpu.VMEM((F_, _NH * _DH), _BF),
            pltpu.VMEM((F_, _NH * _DH), _BF),
            pltpu.VMEM((_V, _H), _BF),
            pltpu.VMEM((_H, _NH * _DH), _BF),
            pltpu.VMEM((1, _NH * _DH), jnp.float32),
        ],
    )(vid3, features, mora3, *weights)
    return out.reshape(B_, ML_, 2, 4)


# speedup vs baseline: 1.0331x; 1.0100x over previous
import jax
import jax.numpy as jnp
from jax.experimental import pallas as pl
from jax.experimental.pallas import tpu as pltpu

_B, _FL, _ML = 16, 2048, 256
_F, _H, _VE, _V = 128, 128, 32, 64
_NH, _DH, _DFF = 4, 32, 512
_BF = jnp.bfloat16
_RPS = 2


def _layer_norm(x, g, b):
    mu = jnp.mean(x, axis=-1, keepdims=True)
    d = x - mu
    var = jnp.mean(d * d, axis=-1, keepdims=True)
    return g * (d * jax.lax.rsqrt(var + 1e-5)) + b


def _bdot(a, b):
    return jnp.dot(a.astype(_BF), b.astype(_BF),
                   preferred_element_type=jnp.float32)


def _body(vid_ref, feat_ref, mora_ref, emb_ref, Wpm_ref, bpm_ref, Wpf_ref,
          bpf_ref, Wq_ref, Wk_ref, Wv_ref, Wo_ref, ln1g_ref, ln1b_ref,
          W1_ref, b1_ref, W2_ref, b2_ref, ln2g_ref, ln2b_ref, Wpost_ref,
          bpost_ref, out_ref, wk_s, wv_s, ew_s, wq_s, bv_s):
    @pl.when(pl.program_id(0) == 0)
    def _():
        wk_s[...] = _bdot(Wpf_ref[...], Wk_ref[...]).astype(_BF)
        wv_s[...] = _bdot(Wpf_ref[...], Wv_ref[...]).astype(_BF)
        ew_s[...] = _bdot(emb_ref[...], Wpm_ref[:_VE, :]).astype(_BF)
        wq_s[...] = (Wq_ref[...] * (1.4426950408889634 / (_DH ** 0.5))
                     ).astype(_BF)
        bv_s[...] = _bdot(bpf_ref[...], Wv_ref[...])

    for r in range(_RPS):
        _row_body(vid_ref, feat_ref, mora_ref, Wpm_ref, bpm_ref, Wo_ref,
                  ln1g_ref, ln1b_ref, W1_ref, b1_ref, W2_ref, b2_ref,
                  ln2g_ref, ln2b_ref, Wpost_ref, bpost_ref, out_ref,
                  wk_s, wv_s, ew_s, wq_s, bv_s, r)


def _row_body(vid_ref, feat_ref, mora_ref, Wpm_ref, bpm_ref, Wo_ref,
              ln1g_ref, ln1b_ref, W1_ref, b1_ref, W2_ref, b2_ref,
              ln2g_ref, ln2b_ref, Wpost_ref, bpost_ref, out_ref,
              wk_s, wv_s, ew_s, wq_s, bv_s, r):
    feat = feat_ref[r].astype(_BF)
    ids = mora_ref[r]
    ohT = (jax.lax.broadcasted_iota(jnp.int32, (_ML, _FL), 0) == ids
           ).astype(_BF)
    cnt = jnp.sum(ohT.astype(jnp.float32), axis=1, keepdims=True)
    ssum = jnp.dot(ohT, feat, preferred_element_type=jnp.float32)
    inv = jnp.where(cnt > 0, 1.0 / jnp.maximum(cnt, 1.0), 0.0)
    mora_feat = ssum * inv

    vids = vid_ref[r]
    voh = (jax.lax.broadcasted_iota(jnp.int32, (_ML, _V), 1) == vids
           ).astype(_BF)
    mhA = jnp.dot(voh, ew_s[...], preferred_element_type=jnp.float32)
    mh = mhA + _bdot(mora_feat, Wpm_ref[_VE:, :]) + bpm_ref[...]

    k = jnp.dot(feat, wk_s[...],
                preferred_element_type=jnp.float32).astype(_BF)
    v = jnp.dot(feat, wv_s[...],
                preferred_element_type=jnp.float32).astype(_BF)
    q = jnp.dot(mh.astype(_BF), wq_s[...],
                preferred_element_type=jnp.float32).astype(_BF)

    ones_col = jnp.ones((_FL, 8), dtype=_BF)
    sls = [slice(h_i * _DH, (h_i + 1) * _DH) for h_i in range(_NH)]
    ss = [jax.lax.dot_general(q[:, sl], k[:, sl], (((1,), (1,)), ((), ())),
                              preferred_element_type=jnp.float32)
          for sl in sls]
    es = [jnp.exp2(s.astype(_BF)) for s in ss]
    vaugs = [jnp.concatenate([v[:, sl], ones_col], axis=1) for sl in sls]
    cds = [jnp.dot(e, va, preferred_element_type=jnp.float32)
           for e, va in zip(es, vaugs)]
    ctxs = [cd[:, :_DH] * (1.0 / cd[:, _DH:_DH + 1]) for cd in cds]
    ctx = jnp.concatenate(ctxs, axis=1) + bv_s[...]

    h = mh + _bdot(ctx, Wo_ref[...])
    h = _layer_norm(h, ln1g_ref[...], ln1b_ref[...])
    ff = jnp.maximum(_bdot(h, W1_ref[...]) + b1_ref[...], 0.0)
    h2 = h + _bdot(ff, W2_ref[...]) + b2_ref[...]
    h2 = _layer_norm(h2, ln2g_ref[...], ln2b_ref[...])
    out_ref[r] = _bdot(h2, Wpost_ref[...]) + bpost_ref[...]


def kernel(vowel_ids, features, mora_index, emb, Wpm, bpm, Wpf, bpf, Wq, Wk,
           Wv, Wo, ln1_g, ln1_b, W1, b1, W2, b2, ln2_g, ln2_b, Wpost, bpost):
    B_, FL_, F_ = features.shape
    ML_ = vowel_ids.shape[1]

    vid3 = vowel_ids.astype(jnp.int32).reshape(B_, ML_, 1)
    mora3 = mora_index.astype(jnp.int32).reshape(B_, 1, FL_)
    row = lambda x: x.reshape(1, -1)

    def full(arr):
        return pl.BlockSpec(arr.shape, lambda b: (0,) * arr.ndim)

    weights = [emb, Wpm, row(bpm), Wpf, row(bpf), Wq, Wk, Wv, Wo,
               row(ln1_g), row(ln1_b), W1, row(b1), W2, row(b2),
               row(ln2_g), row(ln2_b), Wpost, row(bpost)]

    out = pl.pallas_call(
        _body,
        grid=(B_ // _RPS,),
        in_specs=[
            pl.BlockSpec((_RPS, ML_, 1), lambda b: (b, 0, 0)),
            pl.BlockSpec((_RPS, FL_, F_), lambda b: (b, 0, 0)),
            pl.BlockSpec((_RPS, 1, FL_), lambda b: (b, 0, 0)),
        ] + [full(w) for w in weights],
        out_specs=pl.BlockSpec((_RPS, ML_, 8), lambda b: (b, 0, 0)),
        out_shape=jax.ShapeDtypeStruct((B_, ML_, 8), jnp.float32),
        scratch_shapes=[
            pltpu.VMEM((F_, _NH * _DH), _BF),
            pltpu.VMEM((F_, _NH * _DH), _BF),
            pltpu.VMEM((_V, _H), _BF),
            pltpu.VMEM((_H, _NH * _DH), _BF),
            pltpu.VMEM((1, _NH * _DH), jnp.float32),
        ],
    )(vid3, features, mora3, *weights)
    return out.reshape(B_, ML_, 2, 4)


# speedup vs baseline: 1.1415x; 1.1050x over previous
import jax
import jax.numpy as jnp
from jax.experimental import pallas as pl
from jax.experimental.pallas import tpu as pltpu

_B, _FL, _ML = 16, 2048, 256
_F, _H, _VE, _V = 128, 128, 32, 64
_NH, _DH, _DFF = 4, 32, 512
_BF = jnp.bfloat16
_F8 = jnp.float8_e4m3fn
_RPS = 2


def _layer_norm(x, g, b):
    mu = jnp.mean(x, axis=-1, keepdims=True)
    d = x - mu
    var = jnp.mean(d * d, axis=-1, keepdims=True)
    return g * (d * jax.lax.rsqrt(var + 1e-5)) + b


def _bdot(a, b):
    return jnp.dot(a.astype(_BF), b.astype(_BF),
                   preferred_element_type=jnp.float32)


def _body(vid_ref, feat_ref, mora_ref, emb_ref, Wpm_ref, bpm_ref, Wpf_ref,
          bpf_ref, Wq_ref, Wk_ref, Wv_ref, Wo_ref, ln1g_ref, ln1b_ref,
          W1_ref, b1_ref, W2_ref, b2_ref, ln2g_ref, ln2b_ref, Wpost_ref,
          bpost_ref, out_ref, wk_s, wv_s, ew_s, wq_s, bv_s):
    @pl.when(pl.program_id(0) == 0)
    def _():
        wk_s[...] = _bdot(Wpf_ref[...], Wk_ref[...]).astype(_BF)
        wv_s[...] = _bdot(Wpf_ref[...], Wv_ref[...]).astype(_BF)
        ew_s[...] = _bdot(emb_ref[...], Wpm_ref[:_VE, :]).astype(_BF)
        wq_s[...] = (Wq_ref[...] * (1.4426950408889634 / (_DH ** 0.5))
                     ).astype(_BF)
        bv_s[...] = _bdot(bpf_ref[...], Wv_ref[...])

    for r in range(_RPS):
        _row_body(vid_ref, feat_ref, mora_ref, Wpm_ref, bpm_ref, Wo_ref,
                  ln1g_ref, ln1b_ref, W1_ref, b1_ref, W2_ref, b2_ref,
                  ln2g_ref, ln2b_ref, Wpost_ref, bpost_ref, out_ref,
                  wk_s, wv_s, ew_s, wq_s, bv_s, r)


def _row_body(vid_ref, feat_ref, mora_ref, Wpm_ref, bpm_ref, Wo_ref,
              ln1g_ref, ln1b_ref, W1_ref, b1_ref, W2_ref, b2_ref,
              ln2g_ref, ln2b_ref, Wpost_ref, bpost_ref, out_ref,
              wk_s, wv_s, ew_s, wq_s, bv_s, r):
    feat = feat_ref[r].astype(_BF)
    ids = mora_ref[r]
    ohT = (jax.lax.broadcasted_iota(jnp.int32, (_ML, _FL), 0) == ids
           ).astype(_BF)
    cnt = jnp.sum(ohT.astype(jnp.float32), axis=1, keepdims=True)
    ssum = jnp.dot(ohT, feat, preferred_element_type=jnp.float32)
    inv = jnp.where(cnt > 0, 1.0 / jnp.maximum(cnt, 1.0), 0.0)
    mora_feat = ssum * inv

    vids = vid_ref[r]
    voh = (jax.lax.broadcasted_iota(jnp.int32, (_ML, _V), 1) == vids
           ).astype(_BF)
    mhA = jnp.dot(voh, ew_s[...], preferred_element_type=jnp.float32)
    mh = mhA + _bdot(mora_feat, Wpm_ref[_VE:, :]) + bpm_ref[...]

    k = jnp.dot(feat, wk_s[...],
                preferred_element_type=jnp.float32).astype(_BF)
    v = jnp.dot(feat, wv_s[...],
                preferred_element_type=jnp.float32).astype(_BF)
    q = jnp.dot(mh.astype(_BF), wq_s[...],
                preferred_element_type=jnp.float32).astype(_BF)

    ones_col = jnp.ones((_FL, 8), dtype=_BF)
    sls = [slice(h_i * _DH, (h_i + 1) * _DH) for h_i in range(_NH)]
    ss = [jax.lax.dot_general(q[:, sl], k[:, sl], (((1,), (1,)), ((), ())),
                              preferred_element_type=jnp.float32)
          for sl in sls]
    es = [jnp.exp2(s.astype(_BF)).astype(_F8) for s in ss]
    vaugs = [jnp.concatenate([v[:, sl], ones_col], axis=1).astype(_F8)
             for sl in sls]
    cds = [jnp.dot(e, va, preferred_element_type=jnp.float32)
           for e, va in zip(es, vaugs)]
    ctxs = [cd[:, :_DH] * (1.0 / cd[:, _DH:_DH + 1]) for cd in cds]
    ctx = jnp.concatenate(ctxs, axis=1) + bv_s[...]

    h = mh + _bdot(ctx, Wo_ref[...])
    h = _layer_norm(h, ln1g_ref[...], ln1b_ref[...])
    ff = jnp.maximum(_bdot(h, W1_ref[...]) + b1_ref[...], 0.0)
    h2 = h + _bdot(ff, W2_ref[...]) + b2_ref[...]
    h2 = _layer_norm(h2, ln2g_ref[...], ln2b_ref[...])
    out_ref[r] = _bdot(h2, Wpost_ref[...]) + bpost_ref[...]


def kernel(vowel_ids, features, mora_index, emb, Wpm, bpm, Wpf, bpf, Wq, Wk,
           Wv, Wo, ln1_g, ln1_b, W1, b1, W2, b2, ln2_g, ln2_b, Wpost, bpost):
    B_, FL_, F_ = features.shape
    ML_ = vowel_ids.shape[1]

    vid3 = vowel_ids.astype(jnp.int32).reshape(B_, ML_, 1)
    mora3 = mora_index.astype(jnp.int32).reshape(B_, 1, FL_)
    row = lambda x: x.reshape(1, -1)

    def full(arr):
        return pl.BlockSpec(arr.shape, lambda b: (0,) * arr.ndim)

    weights = [emb, Wpm, row(bpm), Wpf, row(bpf), Wq, Wk, Wv, Wo,
               row(ln1_g), row(ln1_b), W1, row(b1), W2, row(b2),
               row(ln2_g), row(ln2_b), Wpost, row(bpost)]

    out = pl.pallas_call(
        _body,
        grid=(B_ // _RPS,),
        in_specs=[
            pl.BlockSpec((_RPS, ML_, 1), lambda b: (b, 0, 0)),
            pl.BlockSpec((_RPS, FL_, F_), lambda b: (b, 0, 0)),
            pl.BlockSpec((_RPS, 1, FL_), lambda b: (b, 0, 0)),
        ] + [full(w) for w in weights],
        out_specs=pl.BlockSpec((_RPS, ML_, 8), lambda b: (b, 0, 0)),
        out_shape=jax.ShapeDtypeStruct((B_, ML_, 8), jnp.float32),
        scratch_shapes=[
            pltpu.VMEM((F_, _NH * _DH), _BF),
            pltpu.VMEM((F_, _NH * _DH), _BF),
            pltpu.VMEM((_V, _H), _BF),
            pltpu.VMEM((_H, _NH * _DH), _BF),
            pltpu.VMEM((1, _NH * _DH), jnp.float32),
        ],
    )(vid3, features, mora3, *weights)
    return out.reshape(B_, ML_, 2, 4)


# speedup vs baseline: 1.1627x; 1.0185x over previous
import jax
import jax.numpy as jnp
from jax.experimental import pallas as pl
from jax.experimental.pallas import tpu as pltpu

_B, _FL, _ML = 16, 2048, 256
_F, _H, _VE, _V = 128, 128, 32, 64
_NH, _DH, _DFF = 4, 32, 512
_BF = jnp.bfloat16
_F8 = jnp.float8_e4m3fn
_RPS = 2


def _layer_norm(x, g, b):
    mu = jnp.mean(x, axis=-1, keepdims=True)
    d = x - mu
    var = jnp.mean(d * d, axis=-1, keepdims=True)
    return g * (d * jax.lax.rsqrt(var + 1e-5)) + b


def _bdot(a, b):
    return jnp.dot(a.astype(_BF), b.astype(_BF),
                   preferred_element_type=jnp.float32)


def _body(vid_ref, feat_ref, mora_ref, emb_ref, Wpm_ref, bpm_ref, Wpf_ref,
          bpf_ref, Wq_ref, Wk_ref, Wv_ref, Wo_ref, ln1g_ref, ln1b_ref,
          W1_ref, b1_ref, W2_ref, b2_ref, ln2g_ref, ln2b_ref, Wpost_ref,
          bpost_ref, out_ref, wk_s, wv_s, ew_s, wq_s, bv_s):
    @pl.when(pl.program_id(0) == 0)
    def _():
        wk_s[...] = _bdot(Wpf_ref[...], Wk_ref[...]).astype(_BF)
        wv_s[...] = _bdot(Wpf_ref[...], Wv_ref[...]).astype(_BF)
        ew_s[...] = _bdot(emb_ref[...], Wpm_ref[:_VE, :]).astype(_BF)
        wq_s[...] = (Wq_ref[...] * (1.4426950408889634 / (_DH ** 0.5))
                     ).astype(_BF)
        bv_s[...] = _bdot(bpf_ref[...], Wv_ref[...])

    for r in range(_RPS):
        _row_body(vid_ref, feat_ref, mora_ref, Wpm_ref, bpm_ref, Wo_ref,
                  ln1g_ref, ln1b_ref, W1_ref, b1_ref, W2_ref, b2_ref,
                  ln2g_ref, ln2b_ref, Wpost_ref, bpost_ref, out_ref,
                  wk_s, wv_s, ew_s, wq_s, bv_s, r)


def _row_body(vid_ref, feat_ref, mora_ref, Wpm_ref, bpm_ref, Wo_ref,
              ln1g_ref, ln1b_ref, W1_ref, b1_ref, W2_ref, b2_ref,
              ln2g_ref, ln2b_ref, Wpost_ref, bpost_ref, out_ref,
              wk_s, wv_s, ew_s, wq_s, bv_s, r):
    feat = feat_ref[r].astype(_BF)
    ids = mora_ref[r]
    ohT = (jax.lax.broadcasted_iota(jnp.int32, (_ML, _FL), 0) == ids
           ).astype(_BF)
    cnt = jnp.sum(ohT.astype(jnp.float32), axis=1, keepdims=True)
    ssum = jnp.dot(ohT, feat, preferred_element_type=jnp.float32)
    inv = jnp.where(cnt > 0, 1.0 / jnp.maximum(cnt, 1.0), 0.0)
    mora_feat = ssum * inv

    vids = vid_ref[r]
    voh = (jax.lax.broadcasted_iota(jnp.int32, (_ML, _V), 1) == vids
           ).astype(_BF)
    mhA = jnp.dot(voh, ew_s[...], preferred_element_type=jnp.float32)
    mh = mhA + _bdot(mora_feat, Wpm_ref[_VE:, :]) + bpm_ref[...]

    k = jnp.dot(feat, wk_s[...],
                preferred_element_type=jnp.float32).astype(_BF)
    v = jnp.dot(feat, wv_s[...],
                preferred_element_type=jnp.float32).astype(_BF)
    q = jnp.dot(mh.astype(_BF), wq_s[...],
                preferred_element_type=jnp.float32).astype(_BF)

    ones_col = jnp.ones((_FL, 8), dtype=_BF)
    sls = [slice(h_i * _DH, (h_i + 1) * _DH) for h_i in range(_NH)]
    q8 = q.astype(_F8)
    k8 = k.astype(_F8)
    ss = [jax.lax.dot_general(q8[:, sl], k8[:, sl], (((1,), (1,)), ((), ())),
                              preferred_element_type=jnp.float32)
          for sl in sls]
    es = [jnp.exp2(s.astype(_BF)).astype(_F8) for s in ss]
    vaugs = [jnp.concatenate([v[:, sl], ones_col], axis=1).astype(_F8)
             for sl in sls]
    cds = [jnp.dot(e, va, preferred_element_type=jnp.float32)
           for e, va in zip(es, vaugs)]
    ctxs = [cd[:, :_DH] * (1.0 / cd[:, _DH:_DH + 1]) for cd in cds]
    ctx = jnp.concatenate(ctxs, axis=1) + bv_s[...]

    h = mh + _bdot(ctx, Wo_ref[...])
    h = _layer_norm(h, ln1g_ref[...], ln1b_ref[...])
    ff = jnp.maximum(_bdot(h, W1_ref[...]) + b1_ref[...], 0.0)
    h2 = h + _bdot(ff, W2_ref[...]) + b2_ref[...]
    h2 = _layer_norm(h2, ln2g_ref[...], ln2b_ref[...])
    out_ref[r] = _bdot(h2, Wpost_ref[...]) + bpost_ref[...]


def kernel(vowel_ids, features, mora_index, emb, Wpm, bpm, Wpf, bpf, Wq, Wk,
           Wv, Wo, ln1_g, ln1_b, W1, b1, W2, b2, ln2_g, ln2_b, Wpost, bpost):
    B_, FL_, F_ = features.shape
    ML_ = vowel_ids.shape[1]

    vid3 = vowel_ids.astype(jnp.int32).reshape(B_, ML_, 1)
    mora3 = mora_index.astype(jnp.int32).reshape(B_, 1, FL_)
    row = lambda x: x.reshape(1, -1)

    def full(arr):
        return pl.BlockSpec(arr.shape, lambda b: (0,) * arr.ndim)

    weights = [emb, Wpm, row(bpm), Wpf, row(bpf), Wq, Wk, Wv, Wo,
               row(ln1_g), row(ln1_b), W1, row(b1), W2, row(b2),
               row(ln2_g), row(ln2_b), Wpost, row(bpost)]

    out = pl.pallas_call(
        _body,
        grid=(B_ // _RPS,),
        in_specs=[
            pl.BlockSpec((_RPS, ML_, 1), lambda b: (b, 0, 0)),
            pl.BlockSpec((_RPS, FL_, F_), lambda b: (b, 0, 0)),
            pl.BlockSpec((_RPS, 1, FL_), lambda b: (b, 0, 0)),
        ] + [full(w) for w in weights],
        out_specs=pl.BlockSpec((_RPS, ML_, 8), lambda b: (b, 0, 0)),
        out_shape=jax.ShapeDtypeStruct((B_, ML_, 8), jnp.float32),
        scratch_shapes=[
            pltpu.VMEM((F_, _NH * _DH), _BF),
            pltpu.VMEM((F_, _NH * _DH), _BF),
            pltpu.VMEM((_V, _H), _BF),
            pltpu.VMEM((_H, _NH * _DH), _BF),
            pltpu.VMEM((1, _NH * _DH), jnp.float32),
        ],
    )(vid3, features, mora3, *weights)
    return out.reshape(B_, ML_, 2, 4)


# speedup vs baseline: 1.1788x; 1.0138x over previous
import jax
import jax.numpy as jnp
from jax.experimental import pallas as pl
from jax.experimental.pallas import tpu as pltpu

_B, _FL, _ML = 16, 2048, 256
_F, _H, _VE, _V = 128, 128, 32, 64
_NH, _DH, _DFF = 4, 32, 512
_BF = jnp.bfloat16
_F8 = jnp.float8_e4m3fn
_RPS = 2


def _layer_norm(x, g, b):
    mu = jnp.mean(x, axis=-1, keepdims=True)
    d = x - mu
    var = jnp.mean(d * d, axis=-1, keepdims=True)
    return g * (d * jax.lax.rsqrt(var + 1e-5)) + b


def _bdot(a, b):
    return jnp.dot(a.astype(_BF), b.astype(_BF),
                   preferred_element_type=jnp.float32)


def _body(vid_ref, feat_ref, mora_ref, emb_ref, Wpm_ref, bpm_ref, Wpf_ref,
          bpf_ref, Wq_ref, Wk_ref, Wv_ref, Wo_ref, ln1g_ref, ln1b_ref,
          W1_ref, b1_ref, W2_ref, b2_ref, ln2g_ref, ln2b_ref, Wpost_ref,
          bpost_ref, out_ref, wk_s, wv_s, ew_s, wq_s, bv_s):
    @pl.when(pl.program_id(0) == 0)
    def _():
        wk_s[...] = _bdot(Wpf_ref[...], Wk_ref[...]).astype(_F8)
        wv_s[...] = _bdot(Wpf_ref[...], Wv_ref[...]).astype(_F8)
        ew_s[...] = _bdot(emb_ref[...], Wpm_ref[:_VE, :]).astype(_BF)
        wq_s[...] = (Wq_ref[...] * (1.4426950408889634 / (_DH ** 0.5))
                     ).astype(_BF)
        bv_s[...] = _bdot(bpf_ref[...], Wv_ref[...])

    for r in range(_RPS):
        _row_body(vid_ref, feat_ref, mora_ref, Wpm_ref, bpm_ref, Wo_ref,
                  ln1g_ref, ln1b_ref, W1_ref, b1_ref, W2_ref, b2_ref,
                  ln2g_ref, ln2b_ref, Wpost_ref, bpost_ref, out_ref,
                  wk_s, wv_s, ew_s, wq_s, bv_s, r)


def _row_body(vid_ref, feat_ref, mora_ref, Wpm_ref, bpm_ref, Wo_ref,
              ln1g_ref, ln1b_ref, W1_ref, b1_ref, W2_ref, b2_ref,
              ln2g_ref, ln2b_ref, Wpost_ref, bpost_ref, out_ref,
              wk_s, wv_s, ew_s, wq_s, bv_s, r):
    feat = feat_ref[r].astype(_BF)
    ids = mora_ref[r]
    ohT = (jax.lax.broadcasted_iota(jnp.int32, (_ML, _FL), 0) == ids
           ).astype(_BF)
    cnt = jnp.sum(ohT.astype(jnp.float32), axis=1, keepdims=True)
    ssum = jnp.dot(ohT, feat, preferred_element_type=jnp.float32)
    inv = jnp.where(cnt > 0, 1.0 / jnp.maximum(cnt, 1.0), 0.0)
    mora_feat = ssum * inv

    vids = vid_ref[r]
    voh = (jax.lax.broadcasted_iota(jnp.int32, (_ML, _V), 1) == vids
           ).astype(_BF)
    mhA = jnp.dot(voh, ew_s[...], preferred_element_type=jnp.float32)
    mh = mhA + _bdot(mora_feat, Wpm_ref[_VE:, :]) + bpm_ref[...]

    feat8 = feat.astype(_F8)
    k = jnp.dot(feat8, wk_s[...],
                preferred_element_type=jnp.float32).astype(_BF)
    v = jnp.dot(feat8, wv_s[...],
                preferred_element_type=jnp.float32).astype(_BF)
    q = jnp.dot(mh.astype(_BF), wq_s[...],
                preferred_element_type=jnp.float32).astype(_BF)

    ones_col = jnp.ones((_FL, 8), dtype=_BF)
    sls = [slice(h_i * _DH, (h_i + 1) * _DH) for h_i in range(_NH)]
    q8 = q.astype(_F8)
    k8 = k.astype(_F8)
    ss = [jax.lax.dot_general(q8[:, sl], k8[:, sl], (((1,), (1,)), ((), ())),
                              preferred_element_type=jnp.float32)
          for sl in sls]
    es = [jnp.exp2(s.astype(_BF)).astype(_F8) for s in ss]
    vaugs = [jnp.concatenate([v[:, sl], ones_col], axis=1).astype(_F8)
             for sl in sls]
    cds = [jnp.dot(e, va, preferred_element_type=jnp.float32)
           for e, va in zip(es, vaugs)]
    ctxs = [cd[:, :_DH] * (1.0 / cd[:, _DH:_DH + 1]) for cd in cds]
    ctx = jnp.concatenate(ctxs, axis=1) + bv_s[...]

    h = mh + _bdot(ctx, Wo_ref[...])
    h = _layer_norm(h, ln1g_ref[...], ln1b_ref[...])
    ff = jnp.maximum(_bdot(h, W1_ref[...]) + b1_ref[...], 0.0)
    h2 = h + _bdot(ff, W2_ref[...]) + b2_ref[...]
    h2 = _layer_norm(h2, ln2g_ref[...], ln2b_ref[...])
    out_ref[r] = _bdot(h2, Wpost_ref[...]) + bpost_ref[...]


def kernel(vowel_ids, features, mora_index, emb, Wpm, bpm, Wpf, bpf, Wq, Wk,
           Wv, Wo, ln1_g, ln1_b, W1, b1, W2, b2, ln2_g, ln2_b, Wpost, bpost):
    B_, FL_, F_ = features.shape
    ML_ = vowel_ids.shape[1]

    vid3 = vowel_ids.astype(jnp.int32).reshape(B_, ML_, 1)
    mora3 = mora_index.astype(jnp.int32).reshape(B_, 1, FL_)
    row = lambda x: x.reshape(1, -1)

    def full(arr):
        return pl.BlockSpec(arr.shape, lambda b: (0,) * arr.ndim)

    weights = [emb, Wpm, row(bpm), Wpf, row(bpf), Wq, Wk, Wv, Wo,
               row(ln1_g), row(ln1_b), W1, row(b1), W2, row(b2),
               row(ln2_g), row(ln2_b), Wpost, row(bpost)]

    out = pl.pallas_call(
        _body,
        grid=(B_ // _RPS,),
        in_specs=[
            pl.BlockSpec((_RPS, ML_, 1), lambda b: (b, 0, 0)),
            pl.BlockSpec((_RPS, FL_, F_), lambda b: (b, 0, 0)),
            pl.BlockSpec((_RPS, 1, FL_), lambda b: (b, 0, 0)),
        ] + [full(w) for w in weights],
        out_specs=pl.BlockSpec((_RPS, ML_, 8), lambda b: (b, 0, 0)),
        out_shape=jax.ShapeDtypeStruct((B_, ML_, 8), jnp.float32),
        scratch_shapes=[
            pltpu.VMEM((F_, _NH * _DH), _F8),
            pltpu.VMEM((F_, _NH * _DH), _F8),
            pltpu.VMEM((_V, _H), _BF),
            pltpu.VMEM((_H, _NH * _DH), _BF),
            pltpu.VMEM((1, _NH * _DH), jnp.float32),
        ],
    )(vid3, features, mora3, *weights)
    return out.reshape(B_, ML_, 2, 4)
